# baseline (speedup 1.0000x reference)
import jax
import jax.numpy as jnp
from jax.experimental import pallas as pl

_RADIUS2 = 64.0
_TI = 2048


def _global_body(pos_ref, feat_ref, gf_ref, wgp_ref, wgf_ref, wgg_ref, bg_ref,
                 wlg_ref, bl_ref, gout_ref, gtp_ref, gtf_ref):
    agg_p = jnp.max(pos_ref[...], axis=2)
    agg_f = jnp.max(feat_ref[...], axis=2)
    g_lin = (jnp.dot(agg_p, wgp_ref[...], preferred_element_type=jnp.float32)
             + jnp.dot(agg_f, wgf_ref[...], preferred_element_type=jnp.float32)
             + jnp.dot(gf_ref[...], wgg_ref[...], preferred_element_type=jnp.float32)
             + bg_ref[...])
    g_out = jnp.tanh(g_lin)
    gout_ref[...] = g_out
    G = wlg_ref.shape[0]
    P = gtp_ref.shape[1]
    gu = g_out[:, G:]
    gterm = (jnp.dot(gu, wlg_ref[...], preferred_element_type=jnp.float32)
             + bl_ref[...])
    gtp_ref[...] = gterm[:, :P, None]
    gtf_ref[...] = gterm[:, P:, None]


def _local_body(psi_ref, phi_ref, posr_ref, featr_ref, pose_ref, featc_ref,
                gtp_ref, gtf_ref,
                app_ref, apf_ref, afp_ref, aff_ref,
                bpp_ref, bpf_ref, bfp_ref, bff_ref,
                outp_ref, outf_ref):
    d2 = jnp.dot(psi_ref[0], phi_ref[0], preferred_element_type=jnp.float32,
                 precision=jax.lax.Precision.HIGHEST)
    maskT = (d2 < _RADIUS2).astype(jnp.float32)
    a4 = jnp.dot(pose_ref[0], maskT, preferred_element_type=jnp.float32)
    sf = jnp.dot(featc_ref[0], maskT, preferred_element_type=jnp.float32)
    p = posr_ref.shape[1]
    cnt = jnp.maximum(a4[p:p + 1, :], 1.0)
    nmp = a4[:p, :] / cnt
    nmf = sf / cnt
    rp = posr_ref[0]
    rf = featr_ref[0]
    linp = (jnp.dot(app_ref[...], rp, preferred_element_type=jnp.float32)
            + jnp.dot(apf_ref[...], rf, preferred_element_type=jnp.float32)
            + jnp.dot(bpp_ref[...], nmp, preferred_element_type=jnp.float32)
            + jnp.dot(bpf_ref[...], nmf, preferred_element_type=jnp.float32)
            + gtp_ref[0])
    linf = (jnp.dot(afp_ref[...], rp, preferred_element_type=jnp.float32)
            + jnp.dot(aff_ref[...], rf, preferred_element_type=jnp.float32)
            + jnp.dot(bfp_ref[...], nmp, preferred_element_type=jnp.float32)
            + jnp.dot(bff_ref[...], nmf, preferred_element_type=jnp.float32)
            + gtf_ref[0])
    outp_ref[0] = jnp.tanh(linp)
    outf_ref[0] = jnp.tanh(linf)


def kernel(positions, features, global_features, W_g, b_g, W_l, b_l):
    B, P, N = positions.shape
    F = features.shape[1]
    G = global_features.shape[1]
    C = P + F
    G2 = 2 * G

    wgp = W_g[:P]
    wgf = W_g[P:C]
    wgg = W_g[C:]
    at = W_l[:C].T
    bt = W_l[C:2 * C].T
    wlg = W_l[2 * C:]
    bg2 = b_g.reshape(1, G2)
    bl2 = b_l.reshape(1, C)
    app, apf = at[:P, :P], at[:P, P:]
    afp, aff = at[P:, :P], at[P:, P:]
    bpp, bpf = bt[:P, :P], bt[:P, P:]
    bfp, bff = bt[P:, :P], bt[P:, P:]

    g_out, gtp, gtf = pl.pallas_call(
        _global_body,
        out_shape=(
            jax.ShapeDtypeStruct((B, G2), jnp.float32),
            jax.ShapeDtypeStruct((B, P, 1), jnp.float32),
            jax.ShapeDtypeStruct((B, F, 1), jnp.float32),
        ),
    )(positions, features, global_features, wgp, wgf, wgg, bg2, wlg, bl2)

    ones_row = jnp.ones((B, 1, N), jnp.float32)
    rn = jnp.sum(positions * positions, axis=1, keepdims=True)
    psi = jnp.concatenate([rn, positions, ones_row], axis=1).transpose(0, 2, 1)
    phi = jnp.concatenate([ones_row, -2.0 * positions, rn], axis=1)
    posext = jnp.concatenate([positions, ones_row], axis=1)

    grid = (B, N // _TI)
    wspec = pl.BlockSpec(None, lambda b, i: (0, 0))
    positions_new, features_new = pl.pallas_call(
        _local_body,
        grid=grid,
        in_specs=[
            pl.BlockSpec((1, N, P + 2), lambda b, i: (b, 0, 0)),
            pl.BlockSpec((1, P + 2, _TI), lambda b, i: (b, 0, i)),
            pl.BlockSpec((1, P, _TI), lambda b, i: (b, 0, i)),
            pl.BlockSpec((1, F, _TI), lambda b, i: (b, 0, i)),
            pl.BlockSpec((1, P + 1, N), lambda b, i: (b, 0, 0)),
            pl.BlockSpec((1, F, N), lambda b, i: (b, 0, 0)),
            pl.BlockSpec((1, P, 1), lambda b, i: (b, 0, 0)),
            pl.BlockSpec((1, F, 1), lambda b, i: (b, 0, 0)),
            pl.BlockSpec((P, P), lambda b, i: (0, 0)),
            pl.BlockSpec((P, F), lambda b, i: (0, 0)),
            pl.BlockSpec((F, P), lambda b, i: (0, 0)),
            pl.BlockSpec((F, F), lambda b, i: (0, 0)),
            pl.BlockSpec((P, P), lambda b, i: (0, 0)),
            pl.BlockSpec((P, F), lambda b, i: (0, 0)),
            pl.BlockSpec((F, P), lambda b, i: (0, 0)),
            pl.BlockSpec((F, F), lambda b, i: (0, 0)),
        ],
        out_specs=(
            pl.BlockSpec((1, P, _TI), lambda b, i: (b, 0, i)),
            pl.BlockSpec((1, F, _TI), lambda b, i: (b, 0, i)),
        ),
        out_shape=(
            jax.ShapeDtypeStruct((B, P, N), jnp.float32),
            jax.ShapeDtypeStruct((B, F, N), jnp.float32),
        ),
    )(psi, phi, positions, features, posext, features, gtp, gtf,
      app, apf, afp, aff, bpp, bpf, bfp, bff)

    global_new = g_out[:, :G]
    return (positions_new, features_new, global_new)


# speedup vs baseline: 1.7825x; 1.7825x over previous
import jax
import jax.numpy as jnp
from jax.experimental import pallas as pl

_RADIUS2 = 64.0
_TI = 2048


def _global_body(pos_ref, feat_ref, gf_ref, wgp_ref, wgf_ref, wgg_ref, bg_ref,
                 wlg_ref, bl_ref, gout_ref, gtp_ref, gtf_ref):
    agg_p = jnp.max(pos_ref[...], axis=2)
    agg_f = jnp.max(feat_ref[...], axis=2)
    g_lin = (jnp.dot(agg_p, wgp_ref[...], preferred_element_type=jnp.float32)
             + jnp.dot(agg_f, wgf_ref[...], preferred_element_type=jnp.float32)
             + jnp.dot(gf_ref[...], wgg_ref[...], preferred_element_type=jnp.float32)
             + bg_ref[...])
    g_out = jnp.tanh(g_lin)
    gout_ref[...] = g_out
    G = wlg_ref.shape[0]
    P = gtp_ref.shape[1]
    gu = g_out[:, G:]
    gterm = (jnp.dot(gu, wlg_ref[...], preferred_element_type=jnp.float32)
             + bl_ref[...])
    gtp_ref[...] = gterm[:, :P, None]
    gtf_ref[...] = gterm[:, P:, None]


def _local_body(psi_ref, phi_ref, posr_ref, featr_ref, pose_ref, featc_ref,
                gtp_ref, gtf_ref,
                app_ref, apf_ref, afp_ref, aff_ref,
                bpp_ref, bpf_ref, bfp_ref, bff_ref,
                outp_ref, outf_ref):
    d2 = jnp.dot(psi_ref[0], phi_ref[0], preferred_element_type=jnp.float32)
    maskT = (d2 < _RADIUS2).astype(jnp.bfloat16)
    a4 = jnp.dot(pose_ref[0], maskT, preferred_element_type=jnp.float32)
    sf = jnp.dot(featc_ref[0], maskT, preferred_element_type=jnp.float32)
    p = posr_ref.shape[1]
    cnt = jnp.maximum(a4[p:p + 1, :], 1.0)
    nmp = a4[:p, :] / cnt
    nmf = sf / cnt
    rp = posr_ref[0]
    rf = featr_ref[0]
    linp = (jnp.dot(app_ref[...], rp, preferred_element_type=jnp.float32)
            + jnp.dot(apf_ref[...], rf, preferred_element_type=jnp.float32)
            + jnp.dot(bpp_ref[...], nmp, preferred_element_type=jnp.float32)
            + jnp.dot(bpf_ref[...], nmf, preferred_element_type=jnp.float32)
            + gtp_ref[0])
    linf = (jnp.dot(afp_ref[...], rp, preferred_element_type=jnp.float32)
            + jnp.dot(aff_ref[...], rf, preferred_element_type=jnp.float32)
            + jnp.dot(bfp_ref[...], nmp, preferred_element_type=jnp.float32)
            + jnp.dot(bff_ref[...], nmf, preferred_element_type=jnp.float32)
            + gtf_ref[0])
    outp_ref[0] = jnp.tanh(linp)
    outf_ref[0] = jnp.tanh(linf)


def kernel(positions, features, global_features, W_g, b_g, W_l, b_l):
    B, P, N = positions.shape
    F = features.shape[1]
    G = global_features.shape[1]
    C = P + F
    G2 = 2 * G

    wgp = W_g[:P]
    wgf = W_g[P:C]
    wgg = W_g[C:]
    at = W_l[:C].T
    bt = W_l[C:2 * C].T
    wlg = W_l[2 * C:]
    bg2 = b_g.reshape(1, G2)
    bl2 = b_l.reshape(1, C)
    app, apf = at[:P, :P], at[:P, P:]
    afp, aff = at[P:, :P], at[P:, P:]
    bpp, bpf = bt[:P, :P], bt[:P, P:]
    bfp, bff = bt[P:, :P], bt[P:, P:]

    g_out, gtp, gtf = pl.pallas_call(
        _global_body,
        out_shape=(
            jax.ShapeDtypeStruct((B, G2), jnp.float32),
            jax.ShapeDtypeStruct((B, P, 1), jnp.float32),
            jax.ShapeDtypeStruct((B, F, 1), jnp.float32),
        ),
    )(positions, features, global_features, wgp, wgf, wgg, bg2, wlg, bl2)

    ones_row = jnp.ones((B, 1, N), jnp.float32)
    rn = jnp.sum(positions * positions, axis=1, keepdims=True)
    psi0 = jnp.concatenate([rn, positions, ones_row], axis=1)
    phi0 = jnp.concatenate([ones_row, -2.0 * positions, rn], axis=1)

    def _split3(x):
        h = x.astype(jnp.bfloat16)
        r = x - h.astype(jnp.float32)
        m = r.astype(jnp.bfloat16)
        l = (r - m.astype(jnp.float32)).astype(jnp.bfloat16)
        return h, m, l

    ph, pm, plo = _split3(psi0)
    fh, fm, flo = _split3(phi0)
    psi = jnp.concatenate([ph, ph, pm, pm, ph, plo], axis=1).transpose(0, 2, 1)
    phi = jnp.concatenate([fh, fm, fh, fm, flo, fh], axis=1)
    posext = jnp.concatenate(
        [positions, ones_row], axis=1).astype(jnp.bfloat16)
    featb = features.astype(jnp.bfloat16)

    grid = (B, N // _TI)
    wspec = pl.BlockSpec(None, lambda b, i: (0, 0))
    positions_new, features_new = pl.pallas_call(
        _local_body,
        grid=grid,
        in_specs=[
            pl.BlockSpec((1, N, 6 * (P + 2)), lambda b, i: (b, 0, 0)),
            pl.BlockSpec((1, 6 * (P + 2), _TI), lambda b, i: (b, 0, i)),
            pl.BlockSpec((1, P, _TI), lambda b, i: (b, 0, i)),
            pl.BlockSpec((1, F, _TI), lambda b, i: (b, 0, i)),
            pl.BlockSpec((1, P + 1, N), lambda b, i: (b, 0, 0)),
            pl.BlockSpec((1, F, N), lambda b, i: (b, 0, 0)),
            pl.BlockSpec((1, P, 1), lambda b, i: (b, 0, 0)),
            pl.BlockSpec((1, F, 1), lambda b, i: (b, 0, 0)),
            pl.BlockSpec((P, P), lambda b, i: (0, 0)),
            pl.BlockSpec((P, F), lambda b, i: (0, 0)),
            pl.BlockSpec((F, P), lambda b, i: (0, 0)),
            pl.BlockSpec((F, F), lambda b, i: (0, 0)),
            pl.BlockSpec((P, P), lambda b, i: (0, 0)),
            pl.BlockSpec((P, F), lambda b, i: (0, 0)),
            pl.BlockSpec((F, P), lambda b, i: (0, 0)),
            pl.BlockSpec((F, F), lambda b, i: (0, 0)),
        ],
        out_specs=(
            pl.BlockSpec((1, P, _TI), lambda b, i: (b, 0, i)),
            pl.BlockSpec((1, F, _TI), lambda b, i: (b, 0, i)),
        ),
        out_shape=(
            jax.ShapeDtypeStruct((B, P, N), jnp.float32),
            jax.ShapeDtypeStruct((B, F, N), jnp.float32),
        ),
    )(psi, phi, positions, features, posext, featb, gtp, gtf,
      app, apf, afp, aff, bpp, bpf, bfp, bff)

    global_new = g_out[:, :G]
    return (positions_new, features_new, global_new)


# speedup vs baseline: 1.8208x; 1.0215x over previous
import jax
import jax.numpy as jnp
from jax.experimental import pallas as pl

_RADIUS2 = 64.0


def _body(xyzT_ref, posr_ref, featr_ref, pose_ref, gfT_ref,
          wgpT_ref, wgfT_ref, wggT_ref, bgT_ref, wlgT_ref, blT_ref,
          app_ref, apf_ref, afp_ref, aff_ref,
          bpp_ref, bpf_ref, bfp_ref, bff_ref,
          gout_ref, outp_ref, outf_ref):
    p = posr_ref.shape[1]
    g = wlgT_ref.shape[1]
    agg_p = jnp.max(posr_ref[0], axis=1, keepdims=True)
    agg_f = jnp.max(featr_ref[0], axis=1, keepdims=True)
    g_lin = (jnp.dot(wgpT_ref[...], agg_p, preferred_element_type=jnp.float32)
             + jnp.dot(wgfT_ref[...], agg_f, preferred_element_type=jnp.float32)
             + jnp.dot(wggT_ref[...], gfT_ref[0], preferred_element_type=jnp.float32)
             + bgT_ref[...])
    g_out = jnp.tanh(g_lin)
    gout_ref[0] = g_out[:g]
    gterm = (jnp.dot(wlgT_ref[...], g_out[g:], preferred_element_type=jnp.float32)
             + blT_ref[...])
    gtp = gterm[:p]
    gtf = gterm[p:]
    xall = xyzT_ref[0, :, 0:1]
    yall = xyzT_ref[0, :, 1:2]
    zall = xyzT_ref[0, :, 2:3]
    xr = posr_ref[0, 0:1, :]
    yr = posr_ref[0, 1:2, :]
    zr = posr_ref[0, 2:3, :]
    dx = xall - xr
    dy = yall - yr
    dz = zall - zr
    d2 = dx * dx + dy * dy + dz * dz
    maskT = (d2 < _RADIUS2).astype(jnp.float32)
    a4 = jnp.dot(pose_ref[0], maskT, preferred_element_type=jnp.float32)
    sf = jnp.dot(featr_ref[0], maskT, preferred_element_type=jnp.float32)
    cnt = jnp.maximum(a4[p:p + 1, :], 1.0)
    nmp = a4[:p, :] / cnt
    nmf = sf / cnt
    rp = posr_ref[0]
    rf = featr_ref[0]
    linp = (jnp.dot(app_ref[...], rp, preferred_element_type=jnp.float32)
            + jnp.dot(apf_ref[...], rf, preferred_element_type=jnp.float32)
            + jnp.dot(bpp_ref[...], nmp, preferred_element_type=jnp.float32)
            + jnp.dot(bpf_ref[...], nmf, preferred_element_type=jnp.float32)
            + gtp)
    linf = (jnp.dot(afp_ref[...], rp, preferred_element_type=jnp.float32)
            + jnp.dot(aff_ref[...], rf, preferred_element_type=jnp.float32)
            + jnp.dot(bfp_ref[...], nmp, preferred_element_type=jnp.float32)
            + jnp.dot(bff_ref[...], nmf, preferred_element_type=jnp.float32)
            + gtf)
    outp_ref[0] = jnp.tanh(linp)
    outf_ref[0] = jnp.tanh(linf)


def kernel(positions, features, global_features, W_g, b_g, W_l, b_l):
    B, P, N = positions.shape
    F = features.shape[1]
    G = global_features.shape[1]
    C = P + F
    G2 = 2 * G

    wgpT = W_g[:P].T
    wgfT = W_g[P:C].T
    wggT = W_g[C:].T
    at = W_l[:C].T
    bt = W_l[C:2 * C].T
    wlgT = W_l[2 * C:].T
    bgT = b_g.reshape(G2, 1)
    blT = b_l.reshape(C, 1)
    app, apf = at[:P, :P], at[:P, P:]
    afp, aff = at[P:, :P], at[P:, P:]
    bpp, bpf = bt[:P, :P], bt[:P, P:]
    bfp, bff = bt[P:, :P], bt[P:, P:]

    xyzT = positions.transpose(0, 2, 1)
    posext = jnp.concatenate(
        [positions, jnp.ones((B, 1, N), jnp.float32)], axis=1)
    gfT = global_features.reshape(B, G, 1)

    ws = lambda a, b: pl.BlockSpec((a, b), lambda i: (0, 0))
    g_out3, positions_new, features_new = pl.pallas_call(
        _body,
        grid=(B,),
        in_specs=[
            pl.BlockSpec((1, N, P), lambda i: (i, 0, 0)),
            pl.BlockSpec((1, P, N), lambda i: (i, 0, 0)),
            pl.BlockSpec((1, F, N), lambda i: (i, 0, 0)),
            pl.BlockSpec((1, P + 1, N), lambda i: (i, 0, 0)),
            pl.BlockSpec((1, G, 1), lambda i: (i, 0, 0)),
            ws(G2, P), ws(G2, F), ws(G2, G), ws(G2, 1), ws(C, G), ws(C, 1),
            ws(P, P), ws(P, F), ws(F, P), ws(F, F),
            ws(P, P), ws(P, F), ws(F, P), ws(F, F),
        ],
        out_specs=(
            pl.BlockSpec((1, G, 1), lambda i: (i, 0, 0)),
            pl.BlockSpec((1, P, N), lambda i: (i, 0, 0)),
            pl.BlockSpec((1, F, N), lambda i: (i, 0, 0)),
        ),
        out_shape=(
            jax.ShapeDtypeStruct((B, G, 1), jnp.float32),
            jax.ShapeDtypeStruct((B, P, N), jnp.float32),
            jax.ShapeDtypeStruct((B, F, N), jnp.float32),
        ),
    )(xyzT, positions, features, posext, gfT,
      wgpT, wgfT, wggT, bgT, wlgT, blT,
      app, apf, afp, aff, bpp, bpf, bfp, bff)

    global_new = g_out3.reshape(B, G)
    return (positions_new, features_new, global_new)


# speedup vs baseline: 2.0772x; 1.1408x over previous
import jax
import jax.numpy as jnp
from jax.experimental import pallas as pl

_RADIUS2 = 64.0
_KT = 4


def _body(xyzT_ref, featT_ref, posr_ref, featr_ref, pose_ref, gfT_ref,
          wgpT_ref, wgfT_ref, wggT_ref, bgT_ref, wlgT_ref, blT_ref,
          app_ref, apf_ref, afp_ref, aff_ref,
          bpp_ref, bpf_ref, bfp_ref, bff_ref,
          gout_ref, outp_ref, outf_ref):
    p = posr_ref.shape[1]
    g = wlgT_ref.shape[1]
    agg_p = jnp.max(posr_ref[0], axis=1, keepdims=True)
    agg_f = jnp.max(featr_ref[0], axis=1, keepdims=True)
    g_lin = (jnp.dot(wgpT_ref[...], agg_p, preferred_element_type=jnp.float32)
             + jnp.dot(wgfT_ref[...], agg_f, preferred_element_type=jnp.float32)
             + jnp.dot(wggT_ref[...], gfT_ref[0], preferred_element_type=jnp.float32)
             + bgT_ref[...])
    g_out = jnp.tanh(g_lin)
    gout_ref[0] = g_out[:g]
    gterm = (jnp.dot(wlgT_ref[...], g_out[g:], preferred_element_type=jnp.float32)
             + blT_ref[...])
    gtp = gterm[:p]
    gtf = gterm[p:]
    n = pose_ref.shape[2]
    ts = n // _KT
    posextT = xyzT_ref[0]
    featT = featT_ref[0]
    pose = pose_ref[0]
    featc = featr_ref[0]
    xr = posr_ref[0, 0:1, :]
    yr = posr_ref[0, 1:2, :]
    zr = posr_ref[0, 2:3, :]
    acc4 = [None] * _KT
    accf = [None] * _KT

    def _add(a, b):
        return b if a is None else a + b

    for r in range(_KT):
        rs = slice(r * ts, (r + 1) * ts)
        xall = posextT[rs, 0:1]
        yall = posextT[rs, 1:2]
        zall = posextT[rs, 2:3]
        for s in range(r + 1):
            cs = slice(s * ts, (s + 1) * ts)
            dx = xall - xr[:, cs]
            dy = yall - yr[:, cs]
            dz = zall - zr[:, cs]
            d2 = dx * dx + dy * dy + dz * dz
            mb = (d2 < _RADIUS2).astype(jnp.float32)
            acc4[s] = _add(acc4[s], jnp.dot(pose[:, rs], mb,
                                            preferred_element_type=jnp.float32))
            accf[s] = _add(accf[s], jnp.dot(featc[:, rs], mb,
                                            preferred_element_type=jnp.float32))
            if r != s:
                t4 = jnp.dot(mb, posextT[cs, :], preferred_element_type=jnp.float32)
                tf = jnp.dot(mb, featT[cs, :], preferred_element_type=jnp.float32)
                acc4[r] = _add(acc4[r], t4.T)
                accf[r] = _add(accf[r], tf.T)
    a4 = jnp.concatenate(acc4, axis=1)
    sf = jnp.concatenate(accf, axis=1)
    cnt = jnp.maximum(a4[p:p + 1, :], 1.0)
    nmp = a4[:p, :] / cnt
    nmf = sf / cnt
    rp = posr_ref[0]
    rf = featr_ref[0]
    linp = (jnp.dot(app_ref[...], rp, preferred_element_type=jnp.float32)
            + jnp.dot(apf_ref[...], rf, preferred_element_type=jnp.float32)
            + jnp.dot(bpp_ref[...], nmp, preferred_element_type=jnp.float32)
            + jnp.dot(bpf_ref[...], nmf, preferred_element_type=jnp.float32)
            + gtp)
    linf = (jnp.dot(afp_ref[...], rp, preferred_element_type=jnp.float32)
            + jnp.dot(aff_ref[...], rf, preferred_element_type=jnp.float32)
            + jnp.dot(bfp_ref[...], nmp, preferred_element_type=jnp.float32)
            + jnp.dot(bff_ref[...], nmf, preferred_element_type=jnp.float32)
            + gtf)
    outp_ref[0] = jnp.tanh(linp)
    outf_ref[0] = jnp.tanh(linf)


def kernel(positions, features, global_features, W_g, b_g, W_l, b_l):
    B, P, N = positions.shape
    F = features.shape[1]
    G = global_features.shape[1]
    C = P + F
    G2 = 2 * G

    wgpT = W_g[:P].T
    wgfT = W_g[P:C].T
    wggT = W_g[C:].T
    at = W_l[:C].T
    bt = W_l[C:2 * C].T
    wlgT = W_l[2 * C:].T
    bgT = b_g.reshape(G2, 1)
    blT = b_l.reshape(C, 1)
    app, apf = at[:P, :P], at[:P, P:]
    afp, aff = at[P:, :P], at[P:, P:]
    bpp, bpf = bt[:P, :P], bt[:P, P:]
    bfp, bff = bt[P:, :P], bt[P:, P:]

    posext = jnp.concatenate(
        [positions, jnp.ones((B, 1, N), jnp.float32)], axis=1)
    posextT = posext.transpose(0, 2, 1)
    featT = features.transpose(0, 2, 1)
    gfT = global_features.reshape(B, G, 1)

    ws = lambda a, b: pl.BlockSpec((a, b), lambda i: (0, 0))
    g_out3, positions_new, features_new = pl.pallas_call(
        _body,
        grid=(B,),
        in_specs=[
            pl.BlockSpec((1, N, P + 1), lambda i: (i, 0, 0)),
            pl.BlockSpec((1, N, F), lambda i: (i, 0, 0)),
            pl.BlockSpec((1, P, N), lambda i: (i, 0, 0)),
            pl.BlockSpec((1, F, N), lambda i: (i, 0, 0)),
            pl.BlockSpec((1, P + 1, N), lambda i: (i, 0, 0)),
            pl.BlockSpec((1, G, 1), lambda i: (i, 0, 0)),
            ws(G2, P), ws(G2, F), ws(G2, G), ws(G2, 1), ws(C, G), ws(C, 1),
            ws(P, P), ws(P, F), ws(F, P), ws(F, F),
            ws(P, P), ws(P, F), ws(F, P), ws(F, F),
        ],
        out_specs=(
            pl.BlockSpec((1, G, 1), lambda i: (i, 0, 0)),
            pl.BlockSpec((1, P, N), lambda i: (i, 0, 0)),
            pl.BlockSpec((1, F, N), lambda i: (i, 0, 0)),
        ),
        out_shape=(
            jax.ShapeDtypeStruct((B, G, 1), jnp.float32),
            jax.ShapeDtypeStruct((B, P, N), jnp.float32),
            jax.ShapeDtypeStruct((B, F, N), jnp.float32),
        ),
    )(posextT, featT, positions, features, posext, gfT,
      wgpT, wgfT, wggT, bgT, wlgT, blT,
      app, apf, afp, aff, bpp, bpf, bfp, bff)

    global_new = g_out3.reshape(B, G)
    return (positions_new, features_new, global_new)


# speedup vs baseline: 3.2921x; 1.5849x over previous
import jax
import jax.numpy as jnp
from jax.experimental import pallas as pl

_RADIUS2 = 64.0
_KT = 4


def _dg0(w, x):
    return jax.lax.dot_general(w, x, (((0,), (0,)), ((), ())),
                               preferred_element_type=jnp.float32)


def _body(pf_ref, nTx_ref, gf_ref, wg_ref, bg_ref, wax_ref, wbx_ref,
          wl_ref, bl_ref, gout_ref, outp_ref, outf_ref):
    p = 3
    n = pf_ref.shape[2]
    g2 = wg_ref.shape[1]
    g = g2 // 2
    pf = pf_ref[0]
    nTx = nTx_ref[0]
    aggs = jnp.max(pf, axis=1, keepdims=True)
    agg_row = jnp.concatenate([aggs[:p].T, aggs[p + 1:].T], axis=1)
    g_lin = (jnp.dot(agg_row, wg_ref[:32, :], preferred_element_type=jnp.float32)
             + jnp.dot(gf_ref[0], wg_ref[32:, :], preferred_element_type=jnp.float32)
             + bg_ref[...])
    g_out = jnp.tanh(g_lin)
    gout_ref[0] = g_out[:, :g]
    gterm = (jnp.dot(g_out[:, g:], wl_ref[64:, :], preferred_element_type=jnp.float32)
             + bl_ref[...])
    gcol = gterm.T
    ts = n // _KT
    xr = pf[0:1, :]
    yr = pf[1:2, :]
    zr = pf[2:3, :]
    acc = [None] * _KT

    def _add(a, b):
        return b if a is None else a + b

    for r in range(_KT):
        rs = slice(r * ts, (r + 1) * ts)
        xall = nTx[rs, 0:1]
        yall = nTx[rs, 1:2]
        zall = nTx[rs, 2:3]
        for s in range(r + 1):
            cs = slice(s * ts, (s + 1) * ts)
            dx = xall - xr[:, cs]
            dy = yall - yr[:, cs]
            dz = zall - zr[:, cs]
            d2 = dx * dx + dy * dy + dz * dz
            mb = (d2 < _RADIUS2).astype(jnp.float32)
            acc[s] = _add(acc[s], jnp.dot(pf[:, rs], mb,
                                          preferred_element_type=jnp.float32))
            if r != s:
                tall = jnp.dot(mb, nTx[cs, :], preferred_element_type=jnp.float32)
                acc[r] = _add(acc[r], tall.T)
    sums = jnp.concatenate(acc, axis=1)
    cnt = jnp.maximum(sums[p:p + 1, :], 1.0)
    nmall = sums / cnt
    lin = (_dg0(wax_ref[...], pf) + _dg0(wbx_ref[...], nmall) + gcol)
    t_all = jnp.tanh(lin)
    outp_ref[0] = t_all[:p]
    outf_ref[0] = t_all[p:]


def kernel(positions, features, global_features, W_g, b_g, W_l, b_l):
    B, P, N = positions.shape
    F = features.shape[1]
    G = global_features.shape[1]
    C = P + F
    G2 = 2 * G

    posefeat = jnp.concatenate(
        [positions, jnp.ones((B, 1, N), jnp.float32), features], axis=1)
    nodesTx = posefeat.transpose(0, 2, 1)
    zrow = jnp.zeros((1, C), jnp.float32)
    wax = jnp.concatenate([W_l[:P], zrow, W_l[P:C]], axis=0)
    wbx = jnp.concatenate([W_l[C:C + P], zrow, W_l[C + P:2 * C]], axis=0)
    gf3 = global_features.reshape(B, 1, G)
    bg2 = b_g.reshape(1, G2)
    bl2 = b_l.reshape(1, C)

    ws = lambda a, b: pl.BlockSpec((a, b), lambda i: (0, 0))
    g_out3, positions_new, features_new = pl.pallas_call(
        _body,
        grid=(B,),
        in_specs=[
            pl.BlockSpec((1, C + 1, N), lambda i: (i, 0, 0)),
            pl.BlockSpec((1, N, C + 1), lambda i: (i, 0, 0)),
            pl.BlockSpec((1, 1, G), lambda i: (i, 0, 0)),
            ws(C + G, G2), ws(1, G2), ws(C + 1, C), ws(C + 1, C),
            ws(2 * C + G, C), ws(1, C),
        ],
        out_specs=(
            pl.BlockSpec((1, 1, G), lambda i: (i, 0, 0)),
            pl.BlockSpec((1, P, N), lambda i: (i, 0, 0)),
            pl.BlockSpec((1, F, N), lambda i: (i, 0, 0)),
        ),
        out_shape=(
            jax.ShapeDtypeStruct((B, 1, G), jnp.float32),
            jax.ShapeDtypeStruct((B, P, N), jnp.float32),
            jax.ShapeDtypeStruct((B, F, N), jnp.float32),
        ),
    )(posefeat, nodesTx, gf3, W_g, bg2, wax, wbx, W_l, bl2)

    global_new = g_out3.reshape(B, G)
    return (positions_new, features_new, global_new)


# speedup vs baseline: 3.4227x; 1.0397x over previous
import jax
import jax.numpy as jnp
from jax.experimental import pallas as pl

_RADIUS2 = 64.0
_KT = 8


def _dg0(w, x):
    return jax.lax.dot_general(w, x, (((0,), (0,)), ((), ())),
                               preferred_element_type=jnp.float32)


def _body(pf_ref, nTx_ref, gf_ref, wg_ref, bg_ref, wax_ref, wbx_ref,
          wl_ref, bl_ref, gout_ref, outp_ref, outf_ref):
    p = 3
    n = pf_ref.shape[2]
    g2 = wg_ref.shape[1]
    g = g2 // 2
    pf = pf_ref[0]
    nTx = nTx_ref[0]
    aggs = jnp.max(pf, axis=1, keepdims=True)
    agg_row = jnp.concatenate([aggs[:p].T, aggs[p + 1:].T], axis=1)
    g_lin = (jnp.dot(agg_row, wg_ref[:32, :], preferred_element_type=jnp.float32)
             + jnp.dot(gf_ref[0], wg_ref[32:, :], preferred_element_type=jnp.float32)
             + bg_ref[...])
    g_out = jnp.tanh(g_lin)
    gout_ref[0] = g_out[:, :g]
    gterm = (jnp.dot(g_out[:, g:], wl_ref[64:, :], preferred_element_type=jnp.float32)
             + bl_ref[...])
    gcol = gterm.T
    ts = n // _KT
    xr = pf[0:1, :]
    yr = pf[1:2, :]
    zr = pf[2:3, :]
    acc = [None] * _KT

    def _add(a, b):
        return b if a is None else a + b

    for r in range(_KT):
        rs = slice(r * ts, (r + 1) * ts)
        xall = nTx[rs, 0:1]
        yall = nTx[rs, 1:2]
        zall = nTx[rs, 2:3]
        for s in range(r + 1):
            cs = slice(s * ts, (s + 1) * ts)
            dx = xall - xr[:, cs]
            dy = yall - yr[:, cs]
            dz = zall - zr[:, cs]
            d2 = dx * dx + dy * dy + dz * dz
            mb = (d2 < _RADIUS2).astype(jnp.float32)
            acc[s] = _add(acc[s], jnp.dot(pf[:, rs], mb,
                                          preferred_element_type=jnp.float32))
            if r != s:
                tall = jnp.dot(mb, nTx[cs, :], preferred_element_type=jnp.float32)
                acc[r] = _add(acc[r], tall.T)
    sums = jnp.concatenate(acc, axis=1)
    cnt = jnp.maximum(sums[p:p + 1, :], 1.0)
    nmall = sums / cnt
    lin = (_dg0(wax_ref[...], pf) + _dg0(wbx_ref[...], nmall) + gcol)
    t_all = jnp.tanh(lin)
    outp_ref[0] = t_all[:p]
    outf_ref[0] = t_all[p:]


def kernel(positions, features, global_features, W_g, b_g, W_l, b_l):
    B, P, N = positions.shape
    F = features.shape[1]
    G = global_features.shape[1]
    C = P + F
    G2 = 2 * G

    posefeat = jnp.concatenate(
        [positions, jnp.ones((B, 1, N), jnp.float32), features], axis=1)
    nodesTx = posefeat.transpose(0, 2, 1)
    zrow = jnp.zeros((1, C), jnp.float32)
    wax = jnp.concatenate([W_l[:P], zrow, W_l[P:C]], axis=0)
    wbx = jnp.concatenate([W_l[C:C + P], zrow, W_l[C + P:2 * C]], axis=0)
    gf3 = global_features.reshape(B, 1, G)
    bg2 = b_g.reshape(1, G2)
    bl2 = b_l.reshape(1, C)

    ws = lambda a, b: pl.BlockSpec((a, b), lambda i: (0, 0))
    g_out3, positions_new, features_new = pl.pallas_call(
        _body,
        grid=(B,),
        in_specs=[
            pl.BlockSpec((1, C + 1, N), lambda i: (i, 0, 0)),
            pl.BlockSpec((1, N, C + 1), lambda i: (i, 0, 0)),
            pl.BlockSpec((1, 1, G), lambda i: (i, 0, 0)),
            ws(C + G, G2), ws(1, G2), ws(C + 1, C), ws(C + 1, C),
            ws(2 * C + G, C), ws(1, C),
        ],
        out_specs=(
            pl.BlockSpec((1, 1, G), lambda i: (i, 0, 0)),
            pl.BlockSpec((1, P, N), lambda i: (i, 0, 0)),
            pl.BlockSpec((1, F, N), lambda i: (i, 0, 0)),
        ),
        out_shape=(
            jax.ShapeDtypeStruct((B, 1, G), jnp.float32),
            jax.ShapeDtypeStruct((B, P, N), jnp.float32),
            jax.ShapeDtypeStruct((B, F, N), jnp.float32),
        ),
    )(posefeat, nodesTx, gf3, W_g, bg2, wax, wbx, W_l, bl2)

    global_new = g_out3.reshape(B, G)
    return (positions_new, features_new, global_new)


# speedup vs baseline: 3.5012x; 1.0229x over previous
import jax
import jax.numpy as jnp
from jax.experimental import pallas as pl

_RADIUS2 = 64.0
_KT = 8


def _dg0(w, x):
    return jax.lax.dot_general(w, x, (((0,), (0,)), ((), ())),
                               preferred_element_type=jnp.float32)


def _body(pf_ref, gf_ref, wg_ref, bg_ref, wax_ref, wbx_ref,
          wl_ref, bl_ref, gout_ref, outp_ref, outf_ref):
    p = 3
    n = pf_ref.shape[2]
    g2 = wg_ref.shape[1]
    g = g2 // 2
    pf = pf_ref[0]
    nTx = pf.T
    aggs = jnp.max(pf, axis=1, keepdims=True)
    agg_row = jnp.concatenate([aggs[:p].T, aggs[p + 1:].T], axis=1)
    g_lin = (jnp.dot(agg_row, wg_ref[:32, :], preferred_element_type=jnp.float32)
             + jnp.dot(gf_ref[0], wg_ref[32:, :], preferred_element_type=jnp.float32)
             + bg_ref[...])
    g_out = jnp.tanh(g_lin)
    gout_ref[0] = g_out[:, :g]
    gterm = (jnp.dot(g_out[:, g:], wl_ref[64:, :], preferred_element_type=jnp.float32)
             + bl_ref[...])
    gcol = gterm.T
    ts = n // _KT
    xr = pf[0:1, :]
    yr = pf[1:2, :]
    zr = pf[2:3, :]
    acc = [None] * _KT

    def _add(a, b):
        return b if a is None else a + b

    for r in range(_KT):
        rs = slice(r * ts, (r + 1) * ts)
        xall = nTx[rs, 0:1]
        yall = nTx[rs, 1:2]
        zall = nTx[rs, 2:3]
        for s in range(r + 1):
            cs = slice(s * ts, (s + 1) * ts)
            dx = xall - xr[:, cs]
            dy = yall - yr[:, cs]
            dz = zall - zr[:, cs]
            d2 = dx * dx + dy * dy + dz * dz
            mb = (d2 < _RADIUS2).astype(jnp.float32)
            acc[s] = _add(acc[s], jnp.dot(pf[:, rs], mb,
                                          preferred_element_type=jnp.float32))
            if r != s:
                tall = jnp.dot(mb, nTx[cs, :], preferred_element_type=jnp.float32)
                acc[r] = _add(acc[r], tall.T)
    sums = jnp.concatenate(acc, axis=1)
    cnt = jnp.maximum(sums[p:p + 1, :], 1.0)
    nmall = sums / cnt
    lin = (_dg0(wax_ref[...], pf) + _dg0(wbx_ref[...], nmall) + gcol)
    t_all = jnp.tanh(lin)
    outp_ref[0] = t_all[:p]
    outf_ref[0] = t_all[p:]


def kernel(positions, features, global_features, W_g, b_g, W_l, b_l):
    B, P, N = positions.shape
    F = features.shape[1]
    G = global_features.shape[1]
    C = P + F
    G2 = 2 * G

    posefeat = jnp.concatenate(
        [positions, jnp.ones((B, 1, N), jnp.float32), features], axis=1)
    zrow = jnp.zeros((1, C), jnp.float32)
    wax = jnp.concatenate([W_l[:P], zrow, W_l[P:C]], axis=0)
    wbx = jnp.concatenate([W_l[C:C + P], zrow, W_l[C + P:2 * C]], axis=0)
    gf3 = global_features.reshape(B, 1, G)
    bg2 = b_g.reshape(1, G2)
    bl2 = b_l.reshape(1, C)

    ws = lambda a, b: pl.BlockSpec((a, b), lambda i: (0, 0))
    g_out3, positions_new, features_new = pl.pallas_call(
        _body,
        grid=(B,),
        in_specs=[
            pl.BlockSpec((1, C + 1, N), lambda i: (i, 0, 0)),
            pl.BlockSpec((1, 1, G), lambda i: (i, 0, 0)),
            ws(C + G, G2), ws(1, G2), ws(C + 1, C), ws(C + 1, C),
            ws(2 * C + G, C), ws(1, C),
        ],
        out_specs=(
            pl.BlockSpec((1, 1, G), lambda i: (i, 0, 0)),
            pl.BlockSpec((1, P, N), lambda i: (i, 0, 0)),
            pl.BlockSpec((1, F, N), lambda i: (i, 0, 0)),
        ),
        out_shape=(
            jax.ShapeDtypeStruct((B, 1, G), jnp.float32),
            jax.ShapeDtypeStruct((B, P, N), jnp.float32),
            jax.ShapeDtypeStruct((B, F, N), jnp.float32),
        ),
    )(posefeat, gf3, W_g, bg2, wax, wbx, W_l, bl2)

    global_new = g_out3.reshape(B, G)
    return (positions_new, features_new, global_new)
